# baseline (speedup 1.0000x reference)
import functools

import jax
import jax.numpy as jnp
from jax import lax
from jax.experimental import pallas as pl
from jax.experimental.pallas import tpu as pltpu
from jax.experimental.pallas import tpu_sc as plsc

_LANES = 16
_NUM_CORES = 2
_NUM_SUBCORES = 16
_NUM_WORKERS = _NUM_CORES * _NUM_SUBCORES


def _exit_points_body(idx_hbm, out_hbm, idx_v, ep_v):
    n_idx = idx_v.shape[0] * 2
    rows_per_w = ep_v.shape[0]
    wid = lax.axis_index("s") * _NUM_CORES + lax.axis_index("c")
    base = wid * rows_per_w

    pltpu.sync_copy(idx_hbm, idx_v)

    @plsc.parallel_loop(0, rows_per_w, step=_LANES, unroll=8)
    def init(j):
        ep_v[pl.ds(j, _LANES)] = jnp.ones((_LANES,), jnp.float32)

    zeros = jnp.zeros((_LANES,), jnp.float32)

    @plsc.parallel_loop(0, n_idx // 2, step=_LANES, unroll=8)
    def scat(j):
        packed = idx_v[pl.ds(j, _LANES)]
        for idx in (packed & 0xFFFF, lax.shift_right_logical(packed, 16)):
            rel = idx - base
            msk = (rel >= 0) & (rel < rows_per_w)
            rel = jnp.where(msk, rel, 0)
            plsc.store_scatter(ep_v, [rel], zeros, mask=msk)

    pltpu.sync_copy(ep_v, out_hbm.at[pl.ds(base, rows_per_w)])


_DN_T = (((0,), (1,)), ((), ()))


def _select_matmul_body(ep_ref, x_ref, wm_ref, we_ref, wg_ref, y_ref, gate_ref):
    x = x_ref[...]
    exited = ep_ref[0] == 0.0
    ym = lax.dot_general(wm_ref[...], x, _DN_T, preferred_element_type=jnp.float32)
    ye = lax.dot_general(we_ref[...], x, _DN_T, preferred_element_type=jnp.float32)
    y_ref[...] = jnp.where(exited, ye, ym)
    g = lax.dot_general(wg_ref[...], x, _DN_T, preferred_element_type=jnp.float32)
    gate_ref[...] = jnp.where(exited, g, jnp.float32(jnp.inf))


def _exit_points_sc(exit_idx, B):
    rows_per_w = B // _NUM_WORKERS
    idx_packed = lax.bitcast_convert_type(
        exit_idx.astype(jnp.int16).reshape(-1, 2), jnp.int32
    )
    sc = pl.kernel(
        _exit_points_body,
        out_type=jax.ShapeDtypeStruct((B,), jnp.float32),
        mesh=plsc.VectorSubcoreMesh(core_axis_name="c", subcore_axis_name="s"),
        scratch_types=[
            pltpu.VMEM((idx_packed.shape[0],), jnp.int32),
            pltpu.VMEM((rows_per_w,), jnp.float32),
        ],
        compiler_params=pltpu.CompilerParams(needs_layout_passes=False),
    )
    return sc(idx_packed)


def _select_matmul_tc(exit_points, X, W_model, W_exit, w_gate, row_block=2048):
    B, d = X.shape
    num_outputs = W_model.shape[1]
    nblk = B // row_block
    grid = (nblk,)
    ep3 = exit_points.reshape(nblk, 1, row_block)
    return pl.pallas_call(
        _select_matmul_body,
        grid=grid,
        in_specs=[
            pl.BlockSpec((1, 1, row_block), lambda i: (i, 0, 0)),
            pl.BlockSpec((row_block, d), lambda i: (i, 0)),
            pl.BlockSpec((d, num_outputs), lambda i: (0, 0)),
            pl.BlockSpec((d, num_outputs), lambda i: (0, 0)),
            pl.BlockSpec((d, 1), lambda i: (0, 0)),
        ],
        out_specs=[
            pl.BlockSpec((num_outputs, row_block), lambda i: (0, i)),
            pl.BlockSpec((1, row_block), lambda i: (0, i)),
        ],
        out_shape=[
            jax.ShapeDtypeStruct((num_outputs, B), X.dtype),
            jax.ShapeDtypeStruct((1, B), X.dtype),
        ],
    )(ep3, X, W_model, W_exit, w_gate)


def kernel(X, exit_idx, W_model, W_exit, w_gate):
    B = X.shape[0]
    exit_points = _exit_points_sc(exit_idx, B)
    y_hat_t, gate_t = _select_matmul_tc(exit_points, X, W_model, W_exit, w_gate)
    return y_hat_t.T, exit_points, gate_t.reshape(B, 1)


# speedup vs baseline: 1.0277x; 1.0277x over previous
import functools

import jax
import jax.numpy as jnp
from jax import lax
from jax.experimental import pallas as pl
from jax.experimental.pallas import tpu as pltpu
from jax.experimental.pallas import tpu_sc as plsc

_LANES = 16
_NUM_CORES = 2
_NUM_SUBCORES = 16
_NUM_WORKERS = _NUM_CORES * _NUM_SUBCORES


def _exit_points_body(idx_hbm, out_hbm, idx_v, ep_v):
    n_idx = idx_v.shape[0]
    rows_per_w = ep_v.shape[0]
    wid = lax.axis_index("s") * _NUM_CORES + lax.axis_index("c")
    base = wid * rows_per_w

    pltpu.sync_copy(idx_hbm, idx_v)

    @plsc.parallel_loop(0, rows_per_w, step=_LANES, unroll=8)
    def init(j):
        ep_v[pl.ds(j, _LANES)] = jnp.ones((_LANES,), jnp.float32)

    zeros = jnp.zeros((_LANES,), jnp.float32)

    @plsc.parallel_loop(0, n_idx, step=_LANES, unroll=8)
    def scat(j):
        idx = idx_v[pl.ds(j, _LANES)]
        rel = idx - base
        msk = (rel >= 0) & (rel < rows_per_w)
        rel = jnp.where(msk, rel, 0)
        plsc.store_scatter(ep_v, [rel], zeros, mask=msk)

    pltpu.sync_copy(ep_v, out_hbm.at[pl.ds(base, rows_per_w)])


_DN_T = (((0,), (1,)), ((), ()))


def _select_matmul_body(ep_ref, x_ref, wm_ref, we_ref, wg_ref, y_ref, gate_ref):
    x = x_ref[...]
    exited = ep_ref[0] == 0.0
    ym = lax.dot_general(wm_ref[...], x, _DN_T, preferred_element_type=jnp.float32)
    ye = lax.dot_general(we_ref[...], x, _DN_T, preferred_element_type=jnp.float32)
    y_ref[...] = jnp.where(exited, ye, ym)
    g = lax.dot_general(wg_ref[...], x, _DN_T, preferred_element_type=jnp.float32)
    gate_ref[...] = jnp.where(exited, g, jnp.float32(jnp.inf))


def _exit_points_sc(exit_idx, B):
    rows_per_w = B // _NUM_WORKERS
    sc = pl.kernel(
        _exit_points_body,
        out_type=jax.ShapeDtypeStruct((B,), jnp.float32),
        mesh=plsc.VectorSubcoreMesh(core_axis_name="c", subcore_axis_name="s"),
        scratch_types=[
            pltpu.VMEM((exit_idx.shape[0],), jnp.int32),
            pltpu.VMEM((rows_per_w,), jnp.float32),
        ],
        compiler_params=pltpu.CompilerParams(needs_layout_passes=False),
    )
    return sc(exit_idx)


def _select_matmul_tc(exit_points, X, W_model, W_exit, w_gate, row_block=2048):
    B, d = X.shape
    num_outputs = W_model.shape[1]
    nblk = B // row_block
    grid = (nblk,)
    ep3 = exit_points.reshape(nblk, 1, row_block)
    return pl.pallas_call(
        _select_matmul_body,
        grid=grid,
        in_specs=[
            pl.BlockSpec((1, 1, row_block), lambda i: (i, 0, 0)),
            pl.BlockSpec((row_block, d), lambda i: (i, 0)),
            pl.BlockSpec((d, num_outputs), lambda i: (0, 0)),
            pl.BlockSpec((d, num_outputs), lambda i: (0, 0)),
            pl.BlockSpec((d, 1), lambda i: (0, 0)),
        ],
        out_specs=[
            pl.BlockSpec((num_outputs, row_block), lambda i: (0, i)),
            pl.BlockSpec((1, row_block), lambda i: (0, i)),
        ],
        out_shape=[
            jax.ShapeDtypeStruct((num_outputs, B), X.dtype),
            jax.ShapeDtypeStruct((1, B), X.dtype),
        ],
    )(ep3, X, W_model, W_exit, w_gate)


def kernel(X, exit_idx, W_model, W_exit, w_gate):
    B = X.shape[0]
    exit_points = _exit_points_sc(exit_idx, B)
    y_hat_t, gate_t = _select_matmul_tc(exit_points, X, W_model, W_exit, w_gate)
    return y_hat_t.T, exit_points, gate_t.reshape(B, 1)
